# baseline (speedup 1.0000x reference)
import functools

import jax
import jax.numpy as jnp
import numpy as np
from jax import lax
from jax.experimental import pallas as pl
from jax.experimental.pallas import tpu as pltpu
from jax.experimental.pallas import tpu_sc as plsc

_SCALE = 1.0 / 16.0
_P = 7
_S = 2
_D = 5
_H = 64
_W = 64
_N = 2
_C = _D * _P * _P
_CP = 256
_R = 512
_NTILES = 32
_CPT = _CP // _NTILES
_RB = _R // 16


def _tile_body(feat_hbm, rois_hbm, ytab_hbm, xtab_hbm, out_hbm,
               feat_v, rois_v, ytab_v, xtab_v, iyd, fyd, ixd, fxd, outbuf):
    nc = 2
    wid = lax.axis_index("s") * nc + lax.axis_index("c")

    cwords = _CPT * _H * _W
    pltpu.sync_copy(feat_hbm.at[0, pl.ds(wid * cwords, cwords)], feat_v.at[0])
    pltpu.sync_copy(feat_hbm.at[1, pl.ds(wid * cwords, cwords)], feat_v.at[1])
    pltpu.sync_copy(rois_hbm, rois_v)
    pltpu.sync_copy(ytab_hbm, ytab_v)
    pltpu.sync_copy(xtab_hbm, xtab_v)

    lane = lax.broadcasted_iota(jnp.int32, (16,), 0)
    ytabv = ytab_v[pl.ds(wid * _CPT, 16)]
    xtabv = xtab_v[pl.ds(wid * _CPT, 16)]

    def rb_body(rb, carry):
        ridx = rb * 16 + lane
        b_f = plsc.load_gather(rois_v, [ridx * 5])
        rx1 = plsc.load_gather(rois_v, [ridx * 5 + 1])
        ry1 = plsc.load_gather(rois_v, [ridx * 5 + 2])
        rx2 = plsc.load_gather(rois_v, [ridx * 5 + 3])
        ry2 = plsc.load_gather(rois_v, [ridx * 5 + 4])
        bvec = b_f.astype(jnp.int32)

        sw = rx1 * _SCALE
        sh = ry1 * _SCALE
        ew = rx2 * _SCALE
        eh = ry2 * _SCALE
        bin_w = jnp.maximum(ew - sw, 0.1) / float(_P)
        bin_h = jnp.maximum(eh - sh, 0.1) / float(_P)

        for p in range(_P):
            for s in range(_S):
                frac = p + (s + 0.5) / _S
                off = p * 64 + s * 32
                yq = sh + frac * bin_h
                vy = jnp.where((yq >= -1.0) & (yq <= float(_H)), 1.0, 0.0)
                yc = jnp.minimum(jnp.maximum(yq, 0.0), float(_H - 1))
                y0 = yc.astype(jnp.int32)
                iyd[pl.ds(off, 16)] = y0 * _W
                iyd[pl.ds(off + 16, 16)] = jnp.minimum(y0 + 1, _H - 1) * _W
                fyd[pl.ds(off, 16)] = yc - y0.astype(jnp.float32)
                fyd[pl.ds(off + 16, 16)] = vy

                xq = sw + frac * bin_w
                vx = jnp.where((xq >= -1.0) & (xq <= float(_W)), 1.0, 0.0)
                xc = jnp.minimum(jnp.maximum(xq, 0.0), float(_W - 1))
                x0 = xc.astype(jnp.int32)
                ixd[pl.ds(off, 16)] = x0
                ixd[pl.ds(off + 16, 16)] = jnp.minimum(x0 + 1, _W - 1)
                fxd[pl.ds(off, 16)] = xc - x0.astype(jnp.float32)
                fxd[pl.ds(off + 16, 16)] = vx

        for cl in range(_CPT):
            yb = ytabv[cl]
            xb = xtabv[cl]
            cbase = cl * (_H * _W)
            acc = jnp.zeros((16,), jnp.float32)
            for s_y in range(_S):
                yo0 = iyd[pl.ds(yb + s_y * 32, 16)] + cbase
                yo1 = iyd[pl.ds(yb + s_y * 32 + 16, 16)] + cbase
                ly = fyd[pl.ds(yb + s_y * 32, 16)]
                vy = fyd[pl.ds(yb + s_y * 32 + 16, 16)]
                for s_x in range(_S):
                    xo0 = ixd[pl.ds(xb + s_x * 32, 16)]
                    xo1 = ixd[pl.ds(xb + s_x * 32 + 16, 16)]
                    lx = fxd[pl.ds(xb + s_x * 32, 16)]
                    vx = fxd[pl.ds(xb + s_x * 32 + 16, 16)]
                    v1 = plsc.load_gather(feat_v, [bvec, yo0 + xo0])
                    v2 = plsc.load_gather(feat_v, [bvec, yo0 + xo1])
                    v3 = plsc.load_gather(feat_v, [bvec, yo1 + xo0])
                    v4 = plsc.load_gather(feat_v, [bvec, yo1 + xo1])
                    top = v1 + lx * (v2 - v1)
                    bot = v3 + lx * (v4 - v3)
                    val = top + ly * (bot - top)
                    acc = acc + val * (vy * vx)
            outbuf[pl.ds(cl * _R + rb * 16, 16)] = acc * (1.0 / (_S * _S))
        return carry

    lax.fori_loop(0, _RB, rb_body, 0)
    pltpu.sync_copy(outbuf, out_hbm.at[pl.ds(wid * (_CPT * _R), _CPT * _R)])


@jax.jit
def _psroi_sc(feat_flat, rois_flat, ytab, xtab):
    mesh = plsc.VectorSubcoreMesh(core_axis_name="c", subcore_axis_name="s")
    f = pl.kernel(
        _tile_body,
        mesh=mesh,
        out_type=jax.ShapeDtypeStruct((_CP * _R,), jnp.float32),
        compiler_params=pltpu.CompilerParams(needs_layout_passes=False),
        scratch_types=[
            pltpu.VMEM((2, _CPT * _H * _W), jnp.float32),
            pltpu.VMEM((_R * 5,), jnp.float32),
            pltpu.VMEM((_CP + 16,), jnp.int32),
            pltpu.VMEM((_CP + 16,), jnp.int32),
            pltpu.VMEM((_P * 64,), jnp.int32),
            pltpu.VMEM((_P * 64,), jnp.float32),
            pltpu.VMEM((_P * 64,), jnp.int32),
            pltpu.VMEM((_P * 64,), jnp.float32),
            pltpu.VMEM((_CPT * _R,), jnp.float32),
        ],
    )
    return f(feat_flat, rois_flat, ytab, xtab)


_ch = np.arange(_CP + 16)
_YTAB = np.ascontiguousarray(((_ch % (_P * _P)) // _P * 64).astype(np.int32))
_XTAB = np.ascontiguousarray((_ch % _P * 64).astype(np.int32))


def kernel(feat, rois):
    n, c, h, w = feat.shape
    featp = jnp.pad(feat, ((0, 0), (0, _CP - c), (0, 0), (0, 0)))
    feat_flat = featp.reshape(n, _CP * h * w)
    out_flat = _psroi_sc(feat_flat, rois.reshape(-1), _YTAB, _XTAB)
    out = out_flat.reshape(_CP, _R)[:c]
    return out.reshape(_D, _P, _P, _R).transpose(3, 0, 1, 2)


# speedup vs baseline: 1.0025x; 1.0025x over previous
import jax
import jax.numpy as jnp
import numpy as np
from jax import lax
from jax.experimental import pallas as pl
from jax.experimental.pallas import tpu as pltpu
from jax.experimental.pallas import tpu_sc as plsc

_SCALE = 1.0 / 16.0
_P = 7
_S = 2
_D = 5
_H = 64
_W = 64
_N = 2
_C = _D * _P * _P
_CP = 256
_R = 512
_NTILES = 32
_CPT = _CP // _NTILES
_RB = _R // 16


def _tile_body(feat_hbm, rois_hbm, ytab_hbm, xtab_hbm, out_hbm,
               feat_v, rois_v, ytab_v, xtab_v, iyd, fyd, ixd, fxd,
               ibv, outbuf, sem0, sem1):
    nc = 2
    wid = lax.axis_index("s") * nc + lax.axis_index("c")

    cwords = _CPT * _H * _W
    cp0 = pltpu.async_copy(feat_hbm.at[0, pl.ds(wid * cwords, cwords)],
                           feat_v.at[0], sem0)
    cp1 = pltpu.async_copy(feat_hbm.at[1, pl.ds(wid * cwords, cwords)],
                           feat_v.at[1], sem1)
    pltpu.sync_copy(rois_hbm, rois_v)
    pltpu.sync_copy(ytab_hbm, ytab_v)
    pltpu.sync_copy(xtab_hbm, xtab_v)

    lane = lax.broadcasted_iota(jnp.int32, (16,), 0)
    ytabv = ytab_v[pl.ds(wid * _CPT, 16)]
    xtabv = xtab_v[pl.ds(wid * _CPT, 16)]

    @plsc.parallel_loop(0, _RB)
    def _(rb):
        ridx = rb * 16 + lane
        b_f = plsc.load_gather(rois_v, [ridx * 5])
        rx1 = plsc.load_gather(rois_v, [ridx * 5 + 1])
        ry1 = plsc.load_gather(rois_v, [ridx * 5 + 2])
        rx2 = plsc.load_gather(rois_v, [ridx * 5 + 3])
        ry2 = plsc.load_gather(rois_v, [ridx * 5 + 4])
        ibv[pl.ds(rb * 16, 16)] = b_f.astype(jnp.int32)

        sw = rx1 * _SCALE
        sh = ry1 * _SCALE
        ew = rx2 * _SCALE
        eh = ry2 * _SCALE
        bin_w = jnp.maximum(ew - sw, 0.1) / float(_P)
        bin_h = jnp.maximum(eh - sh, 0.1) / float(_P)

        for p in range(_P):
            for s in range(_S):
                frac = p + (s + 0.5) / _S
                ofi = rb * 224 + p * 32 + s * 16
                off = rb * 448 + p * 64 + s * 32
                yq = sh + frac * bin_h
                vy = jnp.where((yq >= -1.0) & (yq <= float(_H)), 1.0, 0.0)
                yc = jnp.minimum(jnp.maximum(yq, 0.0), float(_H - 1))
                y0 = yc.astype(jnp.int32)
                iyd[pl.ds(ofi, 16)] = y0 * _W
                fyd[pl.ds(off, 16)] = yc - y0.astype(jnp.float32)
                fyd[pl.ds(off + 16, 16)] = vy

                xq = sw + frac * bin_w
                vx = jnp.where((xq >= -1.0) & (xq <= float(_W)), 1.0, 0.0)
                xc = jnp.minimum(jnp.maximum(xq, 0.0), float(_W - 1))
                x0 = xc.astype(jnp.int32)
                ixd[pl.ds(ofi, 16)] = x0
                fxd[pl.ds(off, 16)] = xc - x0.astype(jnp.float32)
                fxd[pl.ds(off + 16, 16)] = vx

    cp0.wait()
    cp1.wait()

    @plsc.parallel_loop(0, _RB)
    def _(rb):
        bvec = ibv[pl.ds(rb * 16, 16)]
        for cl in range(_CPT):
            yb = ytabv[cl]
            xb = xtabv[cl]
            cbase = cl * (_H * _W)
            acc = jnp.zeros((16,), jnp.float32)
            for s_y in range(_S):
                yo0 = iyd[pl.ds(rb * 224 + yb + s_y * 16, 16)] + cbase
                yo1 = jnp.minimum(yo0 + _W, cbase + (_H - 1) * _W)
                fo = rb * 448 + yb * 2 + s_y * 32
                ly = fyd[pl.ds(fo, 16)]
                vy = fyd[pl.ds(fo + 16, 16)]
                for s_x in range(_S):
                    xo0 = ixd[pl.ds(rb * 224 + xb + s_x * 16, 16)]
                    xo1 = jnp.minimum(xo0 + 1, _W - 1)
                    fo2 = rb * 448 + xb * 2 + s_x * 32
                    lx = fxd[pl.ds(fo2, 16)]
                    vx = fxd[pl.ds(fo2 + 16, 16)]
                    v1 = plsc.load_gather(feat_v, [bvec, yo0 + xo0])
                    v2 = plsc.load_gather(feat_v, [bvec, yo0 + xo1])
                    v3 = plsc.load_gather(feat_v, [bvec, yo1 + xo0])
                    v4 = plsc.load_gather(feat_v, [bvec, yo1 + xo1])
                    top = v1 + lx * (v2 - v1)
                    bot = v3 + lx * (v4 - v3)
                    val = top + ly * (bot - top)
                    acc = acc + val * (vy * vx)
            outbuf[pl.ds(cl * _R + rb * 16, 16)] = acc * (1.0 / (_S * _S))

    pltpu.sync_copy(outbuf, out_hbm.at[pl.ds(wid * (_CPT * _R), _CPT * _R)])


@jax.jit
def _psroi_sc(feat_flat, rois_flat, ytab, xtab):
    mesh = plsc.VectorSubcoreMesh(core_axis_name="c", subcore_axis_name="s")
    f = pl.kernel(
        _tile_body,
        mesh=mesh,
        out_type=jax.ShapeDtypeStruct((_CP * _R,), jnp.float32),
        compiler_params=pltpu.CompilerParams(needs_layout_passes=False),
        scratch_types=[
            pltpu.VMEM((2, _CPT * _H * _W), jnp.float32),
            pltpu.VMEM((_R * 5,), jnp.float32),
            pltpu.VMEM((_CP + 16,), jnp.int32),
            pltpu.VMEM((_CP + 16,), jnp.int32),
            pltpu.VMEM((_RB * 224,), jnp.int32),
            pltpu.VMEM((_RB * 448,), jnp.float32),
            pltpu.VMEM((_RB * 224,), jnp.int32),
            pltpu.VMEM((_RB * 448,), jnp.float32),
            pltpu.VMEM((_R,), jnp.int32),
            pltpu.VMEM((_CPT * _R,), jnp.float32),
            pltpu.SemaphoreType.DMA,
            pltpu.SemaphoreType.DMA,
        ],
    )
    return f(feat_flat, rois_flat, ytab, xtab)


_ch = np.arange(_CP + 16)
_YTAB = np.ascontiguousarray(((_ch % (_P * _P)) // _P * 32).astype(np.int32))
_XTAB = np.ascontiguousarray((_ch % _P * 32).astype(np.int32))


def kernel(feat, rois):
    n, c, h, w = feat.shape
    featp = jnp.pad(feat, ((0, 0), (0, _CP - c), (0, 0), (0, 0)))
    feat_flat = featp.reshape(n, _CP * h * w)
    out_flat = _psroi_sc(feat_flat, rois.reshape(-1), _YTAB, _XTAB)
    out = out_flat.reshape(_CP, _R)[:c]
    return out.reshape(_D, _P, _P, _R).transpose(3, 0, 1, 2)


# speedup vs baseline: 1.1950x; 1.1921x over previous
import jax
import jax.numpy as jnp
import numpy as np
from jax import lax
from jax.experimental import pallas as pl
from jax.experimental.pallas import tpu as pltpu
from jax.experimental.pallas import tpu_sc as plsc

_SCALE = 1.0 / 16.0
_P = 7
_S = 2
_D = 5
_H = 64
_W = 64
_N = 2
_C = _D * _P * _P
_CP = 256
_R = 512
_NTILES = 32
_CPT = _CP // _NTILES
_C0MAX = _C - _CPT
_RB = _R // 16
_HW = _H * _W


def _tile_body(feat_hbm, rois_hbm, ytab_hbm, xtab_hbm, out_hbm,
               feat_v, rois_v, ytab_v, xtab_v, iyd, fyd, ixd, fxd,
               outbuf, sem0, sem1):
    nc = 2
    wid = lax.axis_index("s") * nc + lax.axis_index("c")
    c0 = jnp.minimum(wid * _CPT, _C0MAX)
    d0 = wid * _CPT - c0

    cp0 = pltpu.async_copy(feat_hbm.at[0, pl.ds(c0 * _HW, _CPT * _HW)],
                           feat_v.at[0], sem0)
    cp1 = pltpu.async_copy(feat_hbm.at[1, pl.ds(c0 * _HW, _CPT * _HW)],
                           feat_v.at[1], sem1)
    pltpu.sync_copy(rois_hbm, rois_v)
    pltpu.sync_copy(ytab_hbm, ytab_v)
    pltpu.sync_copy(xtab_hbm, xtab_v)

    lane = lax.broadcasted_iota(jnp.int32, (16,), 0)
    ytabv = ytab_v[pl.ds(wid * _CPT, 16)]
    xtabv = xtab_v[pl.ds(wid * _CPT, 16)]

    cp0.wait()
    cp1.wait()

    def rb_body(rb, carry):
        ridx = rb * 16 + lane
        b_f = plsc.load_gather(rois_v, [ridx * 5])
        rx1 = plsc.load_gather(rois_v, [ridx * 5 + 1])
        ry1 = plsc.load_gather(rois_v, [ridx * 5 + 2])
        rx2 = plsc.load_gather(rois_v, [ridx * 5 + 3])
        ry2 = plsc.load_gather(rois_v, [ridx * 5 + 4])
        bvec = b_f.astype(jnp.int32)

        sw = rx1 * _SCALE
        sh = ry1 * _SCALE
        ew = rx2 * _SCALE
        eh = ry2 * _SCALE
        bin_w = jnp.maximum(ew - sw, 0.1) / float(_P)
        bin_h = jnp.maximum(eh - sh, 0.1) / float(_P)

        for p in range(_P):
            for s in range(_S):
                frac = p + (s + 0.5) / _S
                off = p * 64 + s * 32
                yq = sh + frac * bin_h
                vy = jnp.where((yq >= -1.0) & (yq <= float(_H)), 1.0, 0.0)
                yc = jnp.minimum(jnp.maximum(yq, 0.0), float(_H - 1))
                y0 = yc.astype(jnp.int32)
                iyd[pl.ds(off, 16)] = y0 * _W
                iyd[pl.ds(off + 16, 16)] = jnp.minimum(y0 + 1, _H - 1) * _W
                fyd[pl.ds(off, 16)] = yc - y0.astype(jnp.float32)
                fyd[pl.ds(off + 16, 16)] = vy

                xq = sw + frac * bin_w
                vx = jnp.where((xq >= -1.0) & (xq <= float(_W)), 1.0, 0.0)
                xc = jnp.minimum(jnp.maximum(xq, 0.0), float(_W - 1))
                x0 = xc.astype(jnp.int32)
                ixd[pl.ds(off, 16)] = x0
                ixd[pl.ds(off + 16, 16)] = jnp.minimum(x0 + 1, _W - 1)
                fxd[pl.ds(off, 16)] = xc - x0.astype(jnp.float32)
                fxd[pl.ds(off + 16, 16)] = vx

        for cl in range(_CPT):
            yb = ytabv[cl]
            xb = xtabv[cl]
            cbase = jnp.minimum(cl + d0, _CPT - 1) * _HW
            acc = jnp.zeros((16,), jnp.float32)
            for s_y in range(_S):
                yo0 = iyd[pl.ds(yb + s_y * 32, 16)] + cbase
                yo1 = iyd[pl.ds(yb + s_y * 32 + 16, 16)] + cbase
                ly = fyd[pl.ds(yb + s_y * 32, 16)]
                vy = fyd[pl.ds(yb + s_y * 32 + 16, 16)]
                for s_x in range(_S):
                    xo0 = ixd[pl.ds(xb + s_x * 32, 16)]
                    xo1 = ixd[pl.ds(xb + s_x * 32 + 16, 16)]
                    lx = fxd[pl.ds(xb + s_x * 32, 16)]
                    vx = fxd[pl.ds(xb + s_x * 32 + 16, 16)]
                    v1 = plsc.load_gather(feat_v, [bvec, yo0 + xo0])
                    v2 = plsc.load_gather(feat_v, [bvec, yo0 + xo1])
                    v3 = plsc.load_gather(feat_v, [bvec, yo1 + xo0])
                    v4 = plsc.load_gather(feat_v, [bvec, yo1 + xo1])
                    top = v1 + lx * (v2 - v1)
                    bot = v3 + lx * (v4 - v3)
                    val = top + ly * (bot - top)
                    acc = acc + val * (vy * vx)
            outbuf[pl.ds(cl * _R + rb * 16, 16)] = acc * (1.0 / (_S * _S))
        return carry

    lax.fori_loop(0, _RB, rb_body, 0)
    pltpu.sync_copy(outbuf, out_hbm.at[pl.ds(wid * (_CPT * _R), _CPT * _R)])


@jax.jit
def _psroi_sc(feat_flat, rois_flat, ytab, xtab):
    mesh = plsc.VectorSubcoreMesh(core_axis_name="c", subcore_axis_name="s")
    f = pl.kernel(
        _tile_body,
        mesh=mesh,
        out_type=jax.ShapeDtypeStruct((_CP * _R,), jnp.float32),
        compiler_params=pltpu.CompilerParams(needs_layout_passes=False),
        scratch_types=[
            pltpu.VMEM((2, _CPT * _HW), jnp.float32),
            pltpu.VMEM((_R * 5,), jnp.float32),
            pltpu.VMEM((_CP + 16,), jnp.int32),
            pltpu.VMEM((_CP + 16,), jnp.int32),
            pltpu.VMEM((_P * 64,), jnp.int32),
            pltpu.VMEM((_P * 64,), jnp.float32),
            pltpu.VMEM((_P * 64,), jnp.int32),
            pltpu.VMEM((_P * 64,), jnp.float32),
            pltpu.VMEM((_CPT * _R,), jnp.float32),
            pltpu.SemaphoreType.DMA,
            pltpu.SemaphoreType.DMA,
        ],
    )
    return f(feat_flat, rois_flat, ytab, xtab)


_ch = np.minimum(np.arange(_CP + 16), _C - 1)
_YTAB = np.ascontiguousarray(((_ch % (_P * _P)) // _P * 64).astype(np.int32))
_XTAB = np.ascontiguousarray((_ch % _P * 64).astype(np.int32))


def kernel(feat, rois):
    n, c, h, w = feat.shape
    feat_flat = feat.reshape(n, c * h * w)
    out_flat = _psroi_sc(feat_flat, rois.reshape(-1), _YTAB, _XTAB)
    out = out_flat.reshape(_CP, _R)[:c]
    return out.reshape(_D, _P, _P, _R).transpose(3, 0, 1, 2)


# speedup vs baseline: 1.4863x; 1.2437x over previous
import jax
import jax.numpy as jnp
import numpy as np
from jax import lax
from jax.experimental import pallas as pl
from jax.experimental.pallas import tpu as pltpu
from jax.experimental.pallas import tpu_sc as plsc

_SCALE = 1.0 / 16.0
_P = 7
_S = 2
_D = 5
_H = 64
_W = 64
_N = 2
_C = _D * _P * _P
_CP = 256
_R = 512
_NTILES = 32
_CPT = _CP // _NTILES
_CHP = _CPT // 2
_C0MAX = _C - _CHP
_RB = _R // 16


def _tile_body(feat_hbm, rois_hbm, ytab_hbm, xtab_hbm, out_hbm,
               feat_v, rois_v, ytab_v, xtab_v, iyd, fyd, ixd, fxd,
               ibv, outbuf, sem0, sem1):
    nc = 2
    wid = lax.axis_index("s") * nc + lax.axis_index("c")

    c0a = jnp.minimum(wid * _CPT, _C0MAX)
    cp0 = pltpu.async_copy(feat_hbm.at[0, pl.ds(c0a, _CHP)], feat_v.at[0], sem0)
    cp1 = pltpu.async_copy(feat_hbm.at[1, pl.ds(c0a, _CHP)], feat_v.at[1], sem1)
    pltpu.sync_copy(rois_hbm, rois_v)
    pltpu.sync_copy(ytab_hbm, ytab_v)
    pltpu.sync_copy(xtab_hbm, xtab_v)

    lane = lax.broadcasted_iota(jnp.int32, (16,), 0)
    ytabv = ytab_v[pl.ds(wid * _CPT, 16)]
    xtabv = xtab_v[pl.ds(wid * _CPT, 16)]

    @plsc.parallel_loop(0, _RB)
    def _(rb):
        ridx = rb * 16 + lane
        b_f = plsc.load_gather(rois_v, [ridx * 5])
        rx1 = plsc.load_gather(rois_v, [ridx * 5 + 1])
        ry1 = plsc.load_gather(rois_v, [ridx * 5 + 2])
        rx2 = plsc.load_gather(rois_v, [ridx * 5 + 3])
        ry2 = plsc.load_gather(rois_v, [ridx * 5 + 4])
        ibv[pl.ds(rb * 16, 16)] = b_f.astype(jnp.int32)

        sw = rx1 * _SCALE
        sh = ry1 * _SCALE
        ew = rx2 * _SCALE
        eh = ry2 * _SCALE
        bin_w = jnp.maximum(ew - sw, 0.1) / float(_P)
        bin_h = jnp.maximum(eh - sh, 0.1) / float(_P)

        for p in range(_P):
            for s in range(_S):
                frac = p + (s + 0.5) / _S
                ofi = rb * 224 + p * 32 + s * 16
                off = rb * 448 + p * 64 + s * 32
                yq = sh + frac * bin_h
                vy = jnp.where((yq >= -1.0) & (yq <= float(_H)), 1.0, 0.0)
                yc = jnp.minimum(jnp.maximum(yq, 0.0), float(_H - 1))
                y0 = yc.astype(jnp.int32)
                iyd[pl.ds(ofi, 16)] = y0
                fyd[pl.ds(off, 16)] = yc - y0.astype(jnp.float32)
                fyd[pl.ds(off + 16, 16)] = vy

                xq = sw + frac * bin_w
                vx = jnp.where((xq >= -1.0) & (xq <= float(_W)), 1.0, 0.0)
                xc = jnp.minimum(jnp.maximum(xq, 0.0), float(_W - 1))
                x0 = xc.astype(jnp.int32)
                ixd[pl.ds(ofi, 16)] = x0
                fxd[pl.ds(off, 16)] = xc - x0.astype(jnp.float32)
                fxd[pl.ds(off + 16, 16)] = vx

    cp0.wait()
    cp1.wait()

    for h in range(2):
        c0 = jnp.minimum(wid * _CPT + h * _CHP, _C0MAX)
        d0 = wid * _CPT + h * _CHP - c0
        if h > 0:
            pltpu.sync_copy(feat_hbm.at[0, pl.ds(c0, _CHP)], feat_v.at[0])
            pltpu.sync_copy(feat_hbm.at[1, pl.ds(c0, _CHP)], feat_v.at[1])

        @plsc.parallel_loop(0, _RB)
        def _(rb, h=h, d0=d0):
            bvec = ibv[pl.ds(rb * 16, 16)]
            for cl in range(_CHP):
                yb = ytabv[h * _CHP + cl]
                xb = xtabv[h * _CHP + cl]
                cv = jnp.zeros((16,), jnp.int32) + jnp.minimum(cl + d0,
                                                               _CHP - 1)
                acc = jnp.zeros((16,), jnp.float32)
                for s_y in range(_S):
                    yo0 = iyd[pl.ds(rb * 224 + yb + s_y * 16, 16)]
                    yo1 = jnp.minimum(yo0 + 1, _H - 1)
                    fo = rb * 448 + yb * 2 + s_y * 32
                    ly = fyd[pl.ds(fo, 16)]
                    vy = fyd[pl.ds(fo + 16, 16)]
                    for s_x in range(_S):
                        xo0 = ixd[pl.ds(rb * 224 + xb + s_x * 16, 16)]
                        xo1 = jnp.minimum(xo0 + 1, _W - 1)
                        fo2 = rb * 448 + xb * 2 + s_x * 32
                        lx = fxd[pl.ds(fo2, 16)]
                        vx = fxd[pl.ds(fo2 + 16, 16)]
                        v1 = plsc.load_gather(feat_v, [bvec, cv, yo0, xo0])
                        v2 = plsc.load_gather(feat_v, [bvec, cv, yo0, xo1])
                        v3 = plsc.load_gather(feat_v, [bvec, cv, yo1, xo0])
                        v4 = plsc.load_gather(feat_v, [bvec, cv, yo1, xo1])
                        top = v1 + lx * (v2 - v1)
                        bot = v3 + lx * (v4 - v3)
                        val = top + ly * (bot - top)
                        acc = acc + val * (vy * vx)
                outbuf[pl.ds((h * _CHP + cl) * _R + rb * 16, 16)] = (
                    acc * (1.0 / (_S * _S)))

    pltpu.sync_copy(outbuf, out_hbm.at[pl.ds(wid * (_CPT * _R), _CPT * _R)])


@jax.jit
def _psroi_sc(feat, rois_flat, ytab, xtab):
    mesh = plsc.VectorSubcoreMesh(core_axis_name="c", subcore_axis_name="s")
    f = pl.kernel(
        _tile_body,
        mesh=mesh,
        out_type=jax.ShapeDtypeStruct((_CP * _R,), jnp.float32),
        compiler_params=pltpu.CompilerParams(needs_layout_passes=False,
                                             use_tc_tiling_on_sc=True),
        scratch_types=[
            pltpu.VMEM((2, _CHP, _H, _W), jnp.float32),
            pltpu.VMEM((_R * 5,), jnp.float32),
            pltpu.VMEM((_CP + 16,), jnp.int32),
            pltpu.VMEM((_CP + 16,), jnp.int32),
            pltpu.VMEM((_RB * 224,), jnp.int32),
            pltpu.VMEM((_RB * 448,), jnp.float32),
            pltpu.VMEM((_RB * 224,), jnp.int32),
            pltpu.VMEM((_RB * 448,), jnp.float32),
            pltpu.VMEM((_R,), jnp.int32),
            pltpu.VMEM((_CPT * _R,), jnp.float32),
            pltpu.SemaphoreType.DMA,
            pltpu.SemaphoreType.DMA,
        ],
    )
    return f(feat, rois_flat, ytab, xtab)


_ch = np.minimum(np.arange(_CP + 16), _C - 1)
_YTAB = np.ascontiguousarray(((_ch % (_P * _P)) // _P * 32).astype(np.int32))
_XTAB = np.ascontiguousarray((_ch % _P * 32).astype(np.int32))


def kernel(feat, rois):
    out_flat = _psroi_sc(feat, rois.reshape(-1), _YTAB, _XTAB)
    out = out_flat.reshape(_CP, _R)[:_C]
    return out.reshape(_D, _P, _P, _R).transpose(3, 0, 1, 2)
